# Initial kernel scaffold; baseline (speedup 1.0000x reference)
import jax
import jax.numpy as jnp
from jax.experimental import pallas as pl


def kernel(x, edge_index, edge_attr, W1, b1, W2, b2, ln_weight, ln_bias, gn_weight, gn_bias, gn_mean_scale):
    raise NotImplementedError("write your pallas kernel here")


# speedup vs baseline: 5.0065x; 5.0065x over previous
import functools

import jax
import jax.numpy as jnp
from jax import lax
from jax.experimental import pallas as pl
from jax.experimental.pallas import tpu as pltpu
from jax.experimental.pallas import tpu_sc as plsc

_NC, _NS = 2, 16
_NW = _NC * _NS
_CH = 80
_LANES = 16
_ZB = 128


def _sc_aggregate(x, src3d, dst3d, edge_attr):
    n, d = x.shape
    e = edge_attr.shape[0]
    ept = e // _NW
    nchunk = ept // _CH
    rpt = -(-(n // _NS) // _ZB) * _ZB
    owner = n // rpt
    rem = n % rpt
    ngrp = d // _LANES

    mesh = plsc.VectorSubcoreMesh(core_axis_name="c", subcore_axis_name="s")

    @functools.partial(
        pl.kernel,
        out_type=jax.ShapeDtypeStruct((_NC * n, d), jnp.float32),
        mesh=mesh,
        scratch_types=[
            pltpu.VMEM((nchunk, _CH), jnp.int32),
            pltpu.VMEM((1, _CH), jnp.int32),
            pltpu.VMEM((_CH, d), jnp.float32),
            pltpu.VMEM((_CH, d), jnp.float32),
            pltpu.VMEM_SHARED((_NS * rpt, d), jnp.float32),
            pltpu.SemaphoreType.DMA,
            pltpu.SemaphoreType.DMA,
        ],
    )
    def agg_kernel(x_hbm, src_hbm, dst_hbm, ea_hbm, out_hbm,
                   src_v, dst_v, rows_v, ea_v, acc_sh, sem_x, sem_ea):
        c = lax.axis_index("c")
        s = lax.axis_index("s")
        wid = c * _NS + s

        pltpu.sync_copy(src_hbm.at[wid], src_v)

        def zrow(r, carry):
            for k in range(ngrp):
                ea_v[r, pl.ds(k * _LANES, _LANES)] = jnp.zeros(
                    (_LANES,), jnp.float32)
            return carry
        lax.fori_loop(0, _CH, zrow, 0)
        row0 = s * rpt
        for t in range(rpt // _CH):
            pltpu.sync_copy(ea_v, acc_sh.at[pl.ds(row0 + t * _CH, _CH), :])
        plsc.subcore_barrier()

        base = wid * ept

        def chunk(j, carry):
            off = base + j * _CH
            cp_ea = pltpu.async_copy(ea_hbm.at[pl.ds(off, _CH), :], ea_v, sem_ea)
            cp_x = pltpu.async_copy(x_hbm.at[src_v.at[j]], rows_v, sem_x)
            pltpu.sync_copy(dst_hbm.at[wid, j], dst_v.at[0])
            cp_ea.wait()
            cp_x.wait()

            def arow(r, inner):
                for k in range(ngrp):
                    sl = pl.ds(k * _LANES, _LANES)
                    rows_v[r, sl] = rows_v[r, sl] + ea_v[r, sl]
                return inner
            lax.fori_loop(0, _CH, arow, 0)

            pltpu.sync_copy(rows_v, acc_sh.at[dst_v.at[0]], add=True)
            return carry
        lax.fori_loop(0, nchunk, chunk, 0)

        plsc.subcore_barrier()

        @pl.when(s < owner)
        def _full():
            pltpu.sync_copy(acc_sh.at[pl.ds(row0, rpt), :],
                            out_hbm.at[pl.ds(c * n + row0, rpt), :])

        if rem:
            @pl.when(s == owner)
            def _ragged():
                pltpu.sync_copy(acc_sh.at[pl.ds(row0, rem), :],
                                out_hbm.at[pl.ds(c * n + row0, rem), :])

    return agg_kernel(x, src3d, dst3d, edge_attr)


def _mlp_stats(parts, W1, b1, W2, b2, n, d, br):
    nb = n // br

    def body(p0_ref, p1_ref, w1_ref, b1_ref, w2_ref, b2_ref,
             h_ref, stats_ref, acc_ref):
        i = pl.program_id(0)
        agg = p0_ref[...] + p1_ref[...]
        h1 = jnp.dot(agg, w1_ref[...], preferred_element_type=jnp.float32)
        h1 = jnp.maximum(h1 + b1_ref[...], 0.0)
        h = jnp.dot(h1, w2_ref[...], preferred_element_type=jnp.float32)
        h = h + b2_ref[...]
        h_ref[...] = h

        @pl.when(i == 0)
        def _init():
            acc_ref[...] = jnp.zeros_like(acc_ref)

        acc_ref[0:1, :] += jnp.sum(h, axis=0, keepdims=True)
        acc_ref[1:2, :] += jnp.sum(h * h, axis=0, keepdims=True)

        @pl.when(i == nb - 1)
        def _flush():
            stats_ref[...] = acc_ref[...]

    return pl.pallas_call(
        body,
        grid=(nb,),
        in_specs=[
            pl.BlockSpec((br, d), lambda i: (i, 0)),
            pl.BlockSpec((br, d), lambda i: (i + nb, 0)),
            pl.BlockSpec((d, 2 * d), lambda i: (0, 0)),
            pl.BlockSpec((1, 2 * d), lambda i: (0, 0)),
            pl.BlockSpec((2 * d, d), lambda i: (0, 0)),
            pl.BlockSpec((1, d), lambda i: (0, 0)),
        ],
        out_specs=[
            pl.BlockSpec((br, d), lambda i: (i, 0)),
            pl.BlockSpec((8, d), lambda i: (0, 0)),
        ],
        out_shape=[
            jax.ShapeDtypeStruct((n, d), jnp.float32),
            jax.ShapeDtypeStruct((8, d), jnp.float32),
        ],
        scratch_shapes=[pltpu.VMEM((8, d), jnp.float32)],
    )(parts, parts, W1, b1, W2, b2)


def _norm_gelu_residual(h, x, stats, ln_w, ln_b, gn_w, gn_b, gn_ms, n, d, br):
    nb = n // br
    inv_nd = 1.0 / (n * d)
    inv_n = 1.0 / n

    def body(h_ref, x_ref, st_ref, lnw_ref, lnb_ref, gnw_ref, gnb_ref,
             gms_ref, out_ref):
        s1 = st_ref[0:1, :]
        s2 = st_ref[1:2, :]
        mean = jnp.sum(s1) * inv_nd
        var = jnp.sum(s2) * inv_nd - mean * mean
        a = lnw_ref[...] * lax.rsqrt(var + 1e-5)
        b = lnb_ref[...] - mean * a
        m = a * (s1 * inv_n) + b
        beta = b - m * gms_ref[...]
        v = (a * a * (s2 * inv_n) + 2.0 * a * beta * (s1 * inv_n)
             + beta * beta)
        scale = gnw_ref[...] * lax.rsqrt(v + 1e-5)
        A = a * scale
        B = beta * scale + gnb_ref[...]
        out_ref[...] = jax.nn.gelu(h_ref[...] * A + B) + x_ref[...]

    return pl.pallas_call(
        body,
        grid=(nb,),
        in_specs=[
            pl.BlockSpec((br, d), lambda i: (i, 0)),
            pl.BlockSpec((br, d), lambda i: (i, 0)),
            pl.BlockSpec((8, d), lambda i: (0, 0)),
            pl.BlockSpec((1, d), lambda i: (0, 0)),
            pl.BlockSpec((1, d), lambda i: (0, 0)),
            pl.BlockSpec((1, d), lambda i: (0, 0)),
            pl.BlockSpec((1, d), lambda i: (0, 0)),
            pl.BlockSpec((1, d), lambda i: (0, 0)),
        ],
        out_specs=pl.BlockSpec((br, d), lambda i: (i, 0)),
        out_shape=jax.ShapeDtypeStruct((n, d), jnp.float32),
    )(h, x, stats, ln_w, ln_b, gn_w, gn_b, gn_ms)


def kernel(x, edge_index, edge_attr, W1, b1, W2, b2,
           ln_weight, ln_bias, gn_weight, gn_bias, gn_mean_scale):
    n, d = x.shape
    e = edge_attr.shape[0]
    src3d = edge_index[0].reshape(_NW, e // (_NW * _CH), _CH)
    dst3d = edge_index[1].reshape(_NW, e // (_NW * _CH), _CH)

    parts = _sc_aggregate(x, src3d, dst3d, edge_attr)

    br = 1000
    h, stats = _mlp_stats(parts, W1, b1.reshape(1, -1), W2,
                          b2.reshape(1, -1), n, d, br)
    return _norm_gelu_residual(
        h, x, stats, ln_weight.reshape(1, -1), ln_bias.reshape(1, -1),
        gn_weight.reshape(1, -1), gn_bias.reshape(1, -1),
        gn_mean_scale.reshape(1, -1), n, d, br)
